# baseline (speedup 1.0000x reference)
import functools

import numpy as np
import jax
import jax.numpy as jnp
from jax import lax
from jax.experimental import pallas as pl
from jax.experimental.pallas import tpu as pltpu
from jax.experimental.pallas import tpu_sc as plsc

_PROJ = 128
_BASE = 10000.0
_MAX_LEN = 8192
_SBLK = 512

_LANEI = np.arange(_PROJ)
_THETA = (1.0 / (_BASE ** ((2 * (_LANEI // 2)) / _PROJ))).astype(np.float32)
_MTAB = np.arange(_MAX_LEN, dtype=np.float32)[:, None] * _THETA[None, :]
_CTAB = np.cos(_MTAB).astype(np.float32)
_STAB = (np.sin(_MTAB) * np.where(_LANEI % 2 == 0, -1.0, 1.0)).astype(np.float32)

_NW = 32
_TOK = _MAX_LEN
_PERW = _TOK // _NW
_CHUNK = 128


def _sc_gather_body(sid_hbm, ctab_hbm, stab_hbm, c_out, s_out,
                    idx_v, crows_v, srows_v, sem_c, sem_s):
    w = lax.axis_index("s") * 2 + lax.axis_index("c")
    base = w * _PERW
    for ch in range(_PERW // _CHUNK):
        off = base + ch * _CHUNK
        pltpu.sync_copy(sid_hbm.at[pl.ds(off, _CHUNK)], idx_v)
        cpy_c = pltpu.async_copy(ctab_hbm.at[idx_v], crows_v, sem_c)
        cpy_s = pltpu.async_copy(stab_hbm.at[idx_v], srows_v, sem_s)
        cpy_c.wait()
        cpy_s.wait()
        pltpu.sync_copy(crows_v, c_out.at[pl.ds(off, _CHUNK)])
        pltpu.sync_copy(srows_v, s_out.at[pl.ds(off, _CHUNK)])


def _tc_rope_kernel(c_ref, s_ref, x_ref, o_ref):
    c = c_ref[0]
    s = s_ref[0]
    row = lax.broadcasted_iota(jnp.int32, (_PROJ, _PROJ), 0)
    col = lax.broadcasted_iota(jnp.int32, (_PROJ, _PROJ), 1)
    perm = jnp.where(row == (col ^ 1), 1.0, 0.0).astype(jnp.bfloat16)
    x = x_ref[0]
    x_swap = lax.dot_general(
        x.astype(jnp.bfloat16), perm,
        (((2,), (0,)), ((), ())),
        preferred_element_type=jnp.float32,
    )
    o_ref[0] = c[None] * x + s[None] * x_swap


def kernel(x, seq_id):
    b, h1, h2, seq, proj = x.shape
    heads = h1 * h2
    xf = x.reshape(b, heads, seq, proj)
    sid_flat = seq_id.reshape(b * seq)

    sc_gather = functools.partial(
        pl.kernel,
        mesh=plsc.VectorSubcoreMesh(core_axis_name="c", subcore_axis_name="s"),
        out_type=[
            jax.ShapeDtypeStruct((_TOK, _PROJ), jnp.float32),
            jax.ShapeDtypeStruct((_TOK, _PROJ), jnp.float32),
        ],
        scratch_types=[
            pltpu.VMEM((_CHUNK,), jnp.int32),
            pltpu.VMEM((_CHUNK, _PROJ), jnp.float32),
            pltpu.VMEM((_CHUNK, _PROJ), jnp.float32),
            pltpu.SemaphoreType.DMA,
            pltpu.SemaphoreType.DMA,
        ],
    )(_sc_gather_body)
    c_all, s_all = sc_gather(sid_flat, jnp.asarray(_CTAB), jnp.asarray(_STAB))
    c_all = c_all.reshape(b, seq, proj)
    s_all = s_all.reshape(b, seq, proj)

    nblk = seq // _SBLK
    out = pl.pallas_call(
        _tc_rope_kernel,
        grid=(b, nblk),
        in_specs=[
            pl.BlockSpec((1, _SBLK, proj), lambda i, j: (i, j, 0)),
            pl.BlockSpec((1, _SBLK, proj), lambda i, j: (i, j, 0)),
            pl.BlockSpec((1, heads, _SBLK, proj), lambda i, j: (i, 0, j, 0)),
        ],
        out_specs=pl.BlockSpec((1, heads, _SBLK, proj), lambda i, j: (i, 0, j, 0)),
        out_shape=jax.ShapeDtypeStruct((b, heads, seq, proj), x.dtype),
    )(c_all, s_all, xf)
    return out.reshape(x.shape)


# speedup vs baseline: 1.3377x; 1.3377x over previous
import jax
import jax.numpy as jnp
from jax import lax
from jax.experimental import pallas as pl
from jax.experimental.pallas import tpu as pltpu

_PROJ = 128
_BASE = 10000.0
_SBLK = 512


def _rope_kernel(sid_ref, x_ref, o_ref):
    lane = lax.broadcasted_iota(jnp.int32, (_SBLK, _PROJ), 1)
    pair = (lane // 2).astype(jnp.float32)
    theta = jnp.exp(pair * (-2.0 * jnp.log(_BASE) / _PROJ))
    sid = sid_ref[0, 0, 0, :]
    m = sid[:, None] * theta
    c = jnp.cos(m)
    s = jnp.sin(m)
    row = lax.broadcasted_iota(jnp.int32, (_PROJ, _PROJ), 0)
    col = lax.broadcasted_iota(jnp.int32, (_PROJ, _PROJ), 1)
    pval = jnp.where(col % 2 == 0, -1.0, 1.0)
    perm = jnp.where(row == (col ^ 1), pval, 0.0).astype(jnp.bfloat16)
    x = x_ref[0]
    x_rot = lax.dot_general(
        x.astype(jnp.bfloat16), perm,
        (((2,), (0,)), ((), ())),
        preferred_element_type=jnp.float32,
    )
    o_ref[0] = c[None] * x + s[None] * x_rot


def kernel(x, seq_id):
    b, h1, h2, seq, proj = x.shape
    heads = h1 * h2
    xf = x.reshape(b, heads, seq, proj)
    nblk = seq // _SBLK
    sid = seq_id.reshape(b, nblk, 1, _SBLK).astype(jnp.float32)
    out = pl.pallas_call(
        _rope_kernel,
        grid=(b, nblk),
        in_specs=[
            pl.BlockSpec((1, 1, 1, _SBLK), lambda i, j: (i, j, 0, 0)),
            pl.BlockSpec((1, heads, _SBLK, proj), lambda i, j: (i, 0, j, 0)),
        ],
        out_specs=pl.BlockSpec((1, heads, _SBLK, proj), lambda i, j: (i, 0, j, 0)),
        out_shape=jax.ShapeDtypeStruct((b, heads, seq, proj), x.dtype),
    )(sid, xf)
    return out.reshape(x.shape)
